# baseline (speedup 1.0000x reference)
import functools

import jax
import jax.numpy as jnp
import numpy as np
from jax.experimental import pallas as pl
from jax.experimental.pallas import tpu as pltpu

KS = 128
GF = 0.1
_TR = 512


def _step_table(length: int) -> np.ndarray:
    vals = []
    for ml in range(length + 1):
        max_tokens = max(1, int(round(GF * ml)))
        vals.append(max(1, int(round(ml / max_tokens))))
    return np.asarray(vals, dtype=np.int32)


def _mask_body(seq_ref, table_ref, out_ref, *, nb: int, length: int):
    max_len = seq_ref[0]
    for b in range(1, nb):
        max_len = jnp.maximum(max_len, seq_ref[b])
    step = table_ref[max_len]

    i0 = pl.program_id(0) * _TR
    rows = jax.lax.broadcasted_iota(jnp.int32, (_TR, length), 0) + i0
    cols = jax.lax.broadcasted_iota(jnp.int32, (_TR, length), 1)

    band = jnp.logical_and(cols <= rows, cols >= rows - KS)
    gi = jnp.logical_and(rows < max_len, rows % step == 0)
    gj = jnp.logical_and(cols < max_len, cols % step == 0)
    masked = jnp.logical_or(band, jnp.logical_or(gi, gj))
    out_ref[...] = jnp.where(masked, 0, 1).astype(jnp.int8)


def kernel(x, timestamps, seq_lens):
    length = x.shape[1]
    nb = seq_lens.shape[0]
    table = jnp.asarray(_step_table(length))

    mask8 = pl.pallas_call(
        functools.partial(_mask_body, nb=nb, length=length),
        grid=(length // _TR,),
        in_specs=[
            pl.BlockSpec(memory_space=pltpu.SMEM),
            pl.BlockSpec(memory_space=pltpu.SMEM),
        ],
        out_specs=pl.BlockSpec((_TR, length), lambda i: (i, 0)),
        out_shape=jax.ShapeDtypeStruct((length, length), jnp.int8),
    )(seq_lens.astype(jnp.int32), table)

    return (x, timestamps, mask8.view(jnp.bool_))


# speedup vs baseline: 1.3384x; 1.3384x over previous
import functools

import jax
import jax.numpy as jnp
import numpy as np
from jax.experimental import pallas as pl
from jax.experimental.pallas import tpu as pltpu

KS = 128
GF = 0.1
_TR = 512


def _step_table(length: int) -> np.ndarray:
    vals = []
    for ml in range(length + 1):
        max_tokens = max(1, int(round(GF * ml)))
        vals.append(max(1, int(round(ml / max_tokens))))
    return np.asarray(vals, dtype=np.int32)


def _mask_body(seq_ref, table_ref, out_ref, notg_ref, *, nb: int, length: int):
    max_len = seq_ref[0]
    for b in range(1, nb):
        max_len = jnp.maximum(max_len, seq_ref[b])
    step = table_ref[max_len]

    @pl.when(pl.program_id(0) == 0)
    def _():
        j = jax.lax.broadcasted_iota(jnp.int32, (1, length), 1)
        notg_ref[...] = jnp.where(
            jnp.logical_or(j >= max_len, j % step != 0), 1, 0)

    i0 = pl.program_id(0) * _TR
    rows = jax.lax.broadcasted_iota(jnp.int32, (_TR, length), 0) + i0
    cols = jax.lax.broadcasted_iota(jnp.int32, (_TR, length), 1)
    band = jnp.logical_and(cols <= rows, cols >= rows - KS)
    notgb = jnp.broadcast_to(notg_ref[...], (_TR, length))
    out_ref[...] = jnp.where(band, 0, notgb).astype(jnp.int8)


def kernel(x, timestamps, seq_lens):
    length = x.shape[1]
    nb = seq_lens.shape[0]
    table = jnp.asarray(_step_table(length))

    mask8 = pl.pallas_call(
        functools.partial(_mask_body, nb=nb, length=length),
        grid=(length // _TR,),
        in_specs=[
            pl.BlockSpec(memory_space=pltpu.SMEM),
            pl.BlockSpec(memory_space=pltpu.SMEM),
        ],
        out_specs=pl.BlockSpec((_TR, length), lambda i: (i, 0)),
        out_shape=jax.ShapeDtypeStruct((length, length), jnp.int8),
        scratch_shapes=[pltpu.VMEM((1, length), jnp.int32)],
    )(seq_lens.astype(jnp.int32), table)

    max_len = jnp.max(seq_lens).astype(jnp.int32)
    step = table[max_len]
    p = jnp.arange(length, dtype=jnp.int32)
    notg_rows = jnp.where((p >= max_len) | (p % step != 0), 1, 0).astype(jnp.int8)

    return (x, timestamps, (mask8 * notg_rows[:, None]).view(jnp.bool_))
